# baseline (speedup 1.0000x reference)
import jax
import jax.numpy as jnp
from jax.experimental import pallas as pl
from jax.experimental.pallas import tpu as pltpu

_N = 2048
_NFEAT = 128
_NHID = 8
_NCLASS = 32
_NHEADS = 8
_ALPHA = 0.2
_TILE_C = 256
_NTILES = _N // _TILE_C


def _elu(x):
    return jnp.where(x > 0, x, jnp.exp(x) - 1.0)


def _layer1_kernel(adjT_ref, T_ref, Wh_ref, ah_ref, out_ref,
                   ht_s, cur_s, car_s, cvc_s, cac_s):
    i = pl.program_id(0)

    @pl.when(i == 0)
    def _prep():
        Tm = T_ref[...]
        Tt = Tm.T
        ones = jnp.ones((1, _N), dtype=jnp.bfloat16)
        for hd in range(_NHEADS):
            ht = jnp.dot(Wh_ref[hd].T, Tt, preferred_element_type=jnp.float32,
                         precision=jax.lax.Precision.HIGHEST)
            ht_s[hd] = jnp.concatenate([ht.astype(jnp.bfloat16), ones], axis=0)
        a1 = ah_ref[:, 0, :_NHID]
        a2 = ah_ref[:, 0, _NHID:]
        W1 = jnp.sum(Wh_ref[...] * a1[:, None, :], axis=2).T
        W2 = jnp.sum(Wh_ref[...] * a2[:, None, :], axis=2).T
        U = jnp.dot(Tm, W1, preferred_element_type=jnp.float32,
                    precision=jax.lax.Precision.HIGHEST)
        V = jnp.dot(Tm, W2, preferred_element_type=jnp.float32,
                    precision=jax.lax.Precision.HIGHEST)
        Ut = U.T
        cur_s[...] = (-Ut).astype(jnp.bfloat16)
        car_s[...] = (-_ALPHA * Ut).astype(jnp.bfloat16)
        cvc_s[...] = (-V).astype(jnp.bfloat16)
        cac_s[...] = (-_ALPHA * V).astype(jnp.bfloat16)

    adj_t = adjT_ref[...]
    c0 = i * _TILE_C
    for hd in range(_NHEADS):
        nu1 = cur_s[hd:hd + 1, pl.ds(c0, _TILE_C)]
        nu2 = car_s[hd:hd + 1, pl.ds(c0, _TILE_C)]
        nv1 = cvc_s[:, hd:hd + 1]
        nv2 = cac_s[:, hd:hd + 1]
        arg = jnp.minimum(nv1 + nu1, nv2 + nu2)
        e = jnp.exp(arg) * adj_t
        res = jnp.dot(ht_s[hd], e, preferred_element_type=jnp.float32)
        hp = res[:_NHID, :]
        rowsum = res[_NHID:_NHID + 1, :]
        out_ref[hd * _NHID:(hd + 1) * _NHID, :] = _elu(hp / rowsum)


def _layer2_kernel(adjT_ref, xT_ref, Wo_ref, ao_ref, out_ref,
                   ht_s, cur_s, car_s, cvc_s, cac_s):
    i = pl.program_id(0)

    @pl.when(i == 0)
    def _prep():
        xt = xT_ref[...]
        ht = jnp.dot(Wo_ref[...].T, xt, preferred_element_type=jnp.float32,
                     precision=jax.lax.Precision.HIGHEST)
        ones = jnp.ones((1, _N), dtype=jnp.bfloat16)
        ht_s[...] = jnp.concatenate([ht.astype(jnp.bfloat16), ones], axis=0)
        a1 = ao_ref[0:1, :_NCLASS]
        a2 = ao_ref[0:1, _NCLASS:]
        W1 = jnp.sum(Wo_ref[...] * a1, axis=1, keepdims=True)
        W2 = jnp.sum(Wo_ref[...] * a2, axis=1, keepdims=True)
        Ut = jnp.dot(W1.T, xt, preferred_element_type=jnp.float32,
                     precision=jax.lax.Precision.HIGHEST)
        Vt = jnp.dot(W2.T, xt, preferred_element_type=jnp.float32,
                     precision=jax.lax.Precision.HIGHEST)
        cur_s[...] = (-Ut).astype(jnp.bfloat16)
        car_s[...] = (-_ALPHA * Ut).astype(jnp.bfloat16)
        cvc_s[...] = (-Vt.T).astype(jnp.bfloat16)
        cac_s[...] = (-_ALPHA * Vt.T).astype(jnp.bfloat16)

    adj_t = adjT_ref[...]
    c0 = i * _TILE_C
    nu1 = cur_s[0:1, pl.ds(c0, _TILE_C)]
    nu2 = car_s[0:1, pl.ds(c0, _TILE_C)]
    nv1 = cvc_s[...]
    nv2 = cac_s[...]
    arg = jnp.minimum(nv1 + nu1, nv2 + nu2)
    e = jnp.exp(arg) * adj_t
    res = jnp.dot(ht_s[...], e, preferred_element_type=jnp.float32)
    hp = res[:_NCLASS, :]
    rowsum = res[_NCLASS:_NCLASS + 1, :]
    y = _elu(hp / rowsum)
    m = jnp.max(y, axis=0, keepdims=True)
    z = y - m
    lse = jnp.log(jnp.sum(jnp.exp(z), axis=0, keepdims=True))
    out_ref[...] = (z - lse).T


def kernel(T, adj, W_heads, a_heads, W_out, a_out):
    f32 = jnp.float32
    bf16 = jnp.bfloat16
    adjT = adj.T.astype(bf16)

    x1T = pl.pallas_call(
        _layer1_kernel,
        grid=(_NTILES,),
        in_specs=[
            pl.BlockSpec((_N, _TILE_C), lambda i: (0, i)),
            pl.BlockSpec((_N, _NFEAT), lambda i: (0, 0)),
            pl.BlockSpec((_NHEADS, _NFEAT, _NHID), lambda i: (0, 0, 0)),
            pl.BlockSpec((_NHEADS, 1, 2 * _NHID), lambda i: (0, 0, 0)),
        ],
        out_specs=pl.BlockSpec((_NHEADS * _NHID, _TILE_C), lambda i: (0, i)),
        out_shape=jax.ShapeDtypeStruct((_NHEADS * _NHID, _N), f32),
        scratch_shapes=[
            pltpu.VMEM((_NHEADS, _NHID + 1, _N), bf16),
            pltpu.VMEM((_NHEADS, _N), bf16),
            pltpu.VMEM((_NHEADS, _N), bf16),
            pltpu.VMEM((_N, _NHEADS), bf16),
            pltpu.VMEM((_N, _NHEADS), bf16),
        ],
        compiler_params=pltpu.CompilerParams(
            dimension_semantics=("arbitrary",)),
    )(adjT, T, W_heads, a_heads)

    out = pl.pallas_call(
        _layer2_kernel,
        grid=(_NTILES,),
        in_specs=[
            pl.BlockSpec((_N, _TILE_C), lambda i: (0, i)),
            pl.BlockSpec((_NHEADS * _NHID, _N), lambda i: (0, 0)),
            pl.BlockSpec((_NHEADS * _NHID, _NCLASS), lambda i: (0, 0)),
            pl.BlockSpec((1, 2 * _NCLASS), lambda i: (0, 0)),
        ],
        out_specs=pl.BlockSpec((_TILE_C, _NCLASS), lambda i: (i, 0)),
        out_shape=jax.ShapeDtypeStruct((_N, _NCLASS), f32),
        scratch_shapes=[
            pltpu.VMEM((_NCLASS + 1, _N), bf16),
            pltpu.VMEM((1, _N), bf16),
            pltpu.VMEM((1, _N), bf16),
            pltpu.VMEM((_N, 1), bf16),
            pltpu.VMEM((_N, 1), bf16),
        ],
        compiler_params=pltpu.CompilerParams(
            dimension_semantics=("arbitrary",)),
    )(adjT, x1T, W_out, a_out)
    return out


# speedup vs baseline: 1.3625x; 1.3625x over previous
import jax
import jax.numpy as jnp
from jax.experimental import pallas as pl
from jax.experimental.pallas import tpu as pltpu

_N = 2048
_NFEAT = 128
_NHID = 8
_NCLASS = 32
_NHEADS = 8
_ALPHA = 0.2
_TILE_R = 256
_NTILES = _N // _TILE_R


def _elu(x):
    return jnp.where(x > 0, x, jnp.exp(x) - 1.0)


def _gat_kernel(adj_ref, T_ref, Wh_ref, ah_ref, Wo_ref, ao_ref, out_ref,
                adjb_s, h1_s, cu1_s, cv1_s, x1_s, h2_s, cu2_s, cv2_s):
    l = pl.program_id(0)
    t = pl.program_id(1)
    r0 = t * _TILE_R

    @pl.when(jnp.logical_and(l == 0, t == 0))
    def _prep1():
        adjb_s[...] = adj_ref[...].astype(jnp.bfloat16)
        Tm = T_ref[...]
        ones = jnp.ones((_N, 1), dtype=jnp.bfloat16)
        for hd in range(_NHEADS):
            h = jnp.dot(Tm, Wh_ref[hd], preferred_element_type=jnp.float32,
                        precision=jax.lax.Precision.HIGHEST)
            h1_s[hd] = jnp.concatenate([h.astype(jnp.bfloat16), ones], axis=1)
        a1 = ah_ref[:, 0, :_NHID]
        a2 = ah_ref[:, 0, _NHID:]
        W1 = jnp.sum(Wh_ref[...] * a1[:, None, :], axis=2).T
        W2 = jnp.sum(Wh_ref[...] * a2[:, None, :], axis=2).T
        U = jnp.dot(Tm, W1, preferred_element_type=jnp.float32,
                    precision=jax.lax.Precision.HIGHEST)
        V = jnp.dot(Tm, W2, preferred_element_type=jnp.float32,
                    precision=jax.lax.Precision.HIGHEST)
        cu1_s[...] = (-U).astype(jnp.bfloat16)
        cv1_s[...] = (-V.T).astype(jnp.bfloat16)

    @pl.when(jnp.logical_and(l == 1, t == 0))
    def _prep2():
        xm = x1_s[...]
        h = jnp.dot(xm, Wo_ref[...], preferred_element_type=jnp.float32,
                    precision=jax.lax.Precision.HIGHEST)
        ones = jnp.ones((_N, 1), dtype=jnp.bfloat16)
        h2_s[...] = jnp.concatenate([h.astype(jnp.bfloat16), ones], axis=1)
        a1 = ao_ref[0:1, :_NCLASS]
        a2 = ao_ref[0:1, _NCLASS:]
        W1 = jnp.sum(Wo_ref[...] * a1, axis=1, keepdims=True)
        W2 = jnp.sum(Wo_ref[...] * a2, axis=1, keepdims=True)
        U = jnp.dot(xm, W1, preferred_element_type=jnp.float32,
                    precision=jax.lax.Precision.HIGHEST)
        V = jnp.dot(xm, W2, preferred_element_type=jnp.float32,
                    precision=jax.lax.Precision.HIGHEST)
        cu2_s[...] = (-U).astype(jnp.bfloat16)
        cv2_s[...] = (-V.T).astype(jnp.bfloat16)

    @pl.when(l == 0)
    def _layer1_tile():
        adj_t = adjb_s[pl.ds(r0, _TILE_R), :]
        for hd in range(_NHEADS):
            nu = cu1_s[pl.ds(r0, _TILE_R), hd:hd + 1]
            nv = cv1_s[hd:hd + 1, :]
            s = nu + nv
            arg = jnp.minimum(s, _ALPHA * s)
            e = jnp.exp(arg) * adj_t
            res = jnp.dot(e, h1_s[hd], preferred_element_type=jnp.float32)
            hp = res[:, :_NHID]
            rowsum = res[:, _NHID:_NHID + 1]
            x1_s[pl.ds(r0, _TILE_R), hd * _NHID:(hd + 1) * _NHID] = (
                _elu(hp / rowsum))

    @pl.when(l == 1)
    def _layer2_tile():
        adj_t = adjb_s[pl.ds(r0, _TILE_R), :]
        nu = cu2_s[pl.ds(r0, _TILE_R), :]
        nv = cv2_s[...]
        s = nu + nv
        arg = jnp.minimum(s, _ALPHA * s)
        e = jnp.exp(arg) * adj_t
        res = jnp.dot(e, h2_s[...], preferred_element_type=jnp.float32)
        hp = res[:, :_NCLASS]
        rowsum = res[:, _NCLASS:_NCLASS + 1]
        y = _elu(hp / rowsum)
        m = jnp.max(y, axis=1, keepdims=True)
        z = y - m
        lse = jnp.log(jnp.sum(jnp.exp(z), axis=1, keepdims=True))
        out_ref[...] = z - lse


def kernel(T, adj, W_heads, a_heads, W_out, a_out):
    f32 = jnp.float32
    bf16 = jnp.bfloat16

    out = pl.pallas_call(
        _gat_kernel,
        grid=(2, _NTILES),
        in_specs=[
            pl.BlockSpec((_N, _N), lambda l, t: (0, 0)),
            pl.BlockSpec((_N, _NFEAT), lambda l, t: (0, 0)),
            pl.BlockSpec((_NHEADS, _NFEAT, _NHID), lambda l, t: (0, 0, 0)),
            pl.BlockSpec((_NHEADS, 1, 2 * _NHID), lambda l, t: (0, 0, 0)),
            pl.BlockSpec((_NHEADS * _NHID, _NCLASS), lambda l, t: (0, 0)),
            pl.BlockSpec((1, 2 * _NCLASS), lambda l, t: (0, 0)),
        ],
        out_specs=pl.BlockSpec((_TILE_R, _NCLASS), lambda l, t: (t, 0)),
        out_shape=jax.ShapeDtypeStruct((_N, _NCLASS), f32),
        scratch_shapes=[
            pltpu.VMEM((_N, _N), bf16),
            pltpu.VMEM((_NHEADS, _N, _NHID + 1), bf16),
            pltpu.VMEM((_N, _NHEADS), bf16),
            pltpu.VMEM((_NHEADS, _N), bf16),
            pltpu.VMEM((_N, _NHEADS * _NHID), f32),
            pltpu.VMEM((_N, _NCLASS + 1), bf16),
            pltpu.VMEM((_N, 1), bf16),
            pltpu.VMEM((1, _N), bf16),
        ],
        compiler_params=pltpu.CompilerParams(
            dimension_semantics=("arbitrary", "arbitrary")),
    )(adj, T, W_heads, a_heads, W_out, a_out)
    return out


# speedup vs baseline: 1.4136x; 1.0376x over previous
import jax
import jax.numpy as jnp
from jax.experimental import pallas as pl
from jax.experimental.pallas import tpu as pltpu

_N = 2048
_NFEAT = 128
_NHID = 8
_NCLASS = 32
_NHEADS = 8
_ALPHA = 0.2
_LOG2E = 1.4426950408889634
_TILE_R = 256
_NTILES = _N // _TILE_R


def _elu(x):
    return jnp.where(x > 0, x, jnp.exp(x) - 1.0)


def _gat_kernel(adj_ref, T_ref, Wh_ref, ah_ref, Wo_ref, ao_ref, out_ref,
                adjb_s, h1_s, cu1_s, cv1_s, x1_s, h2_s, cu2_s, cv2_s):
    l = pl.program_id(0)
    t = pl.program_id(1)
    r0 = t * _TILE_R

    @pl.when(jnp.logical_and(l == 0, t == 0))
    def _prep1():
        adjb_s[...] = adj_ref[...].astype(jnp.bfloat16)
        Tm = T_ref[...]
        ones = jnp.ones((_N, 1), dtype=jnp.bfloat16)
        for hd in range(_NHEADS):
            h = jnp.dot(Tm, Wh_ref[hd], preferred_element_type=jnp.float32,
                        precision=jax.lax.Precision.HIGHEST)
            h1_s[hd] = jnp.concatenate([h.astype(jnp.bfloat16), ones], axis=1)
        a1 = ah_ref[:, 0, :_NHID]
        a2 = ah_ref[:, 0, _NHID:]
        W1 = jnp.sum(Wh_ref[...] * a1[:, None, :], axis=2).T
        W2 = jnp.sum(Wh_ref[...] * a2[:, None, :], axis=2).T
        U = jnp.dot(Tm, W1, preferred_element_type=jnp.float32,
                    precision=jax.lax.Precision.HIGHEST)
        V = jnp.dot(Tm, W2, preferred_element_type=jnp.float32,
                    precision=jax.lax.Precision.HIGHEST)
        cu1_s[...] = (-_LOG2E * U).astype(jnp.bfloat16)
        cv1_s[...] = (-_LOG2E * V.T).astype(jnp.bfloat16)

    @pl.when(jnp.logical_and(l == 1, t == 0))
    def _prep2():
        xm = x1_s[...]
        h = jnp.dot(xm, Wo_ref[...], preferred_element_type=jnp.float32,
                    precision=jax.lax.Precision.HIGHEST)
        ones = jnp.ones((_N, 1), dtype=jnp.bfloat16)
        h2_s[...] = jnp.concatenate([h.astype(jnp.bfloat16), ones], axis=1)
        a1 = ao_ref[0:1, :_NCLASS]
        a2 = ao_ref[0:1, _NCLASS:]
        W1 = jnp.sum(Wo_ref[...] * a1, axis=1, keepdims=True)
        W2 = jnp.sum(Wo_ref[...] * a2, axis=1, keepdims=True)
        U = jnp.dot(xm, W1, preferred_element_type=jnp.float32,
                    precision=jax.lax.Precision.HIGHEST)
        V = jnp.dot(xm, W2, preferred_element_type=jnp.float32,
                    precision=jax.lax.Precision.HIGHEST)
        cu2_s[...] = (-_LOG2E * U).astype(jnp.bfloat16)
        cv2_s[...] = (-_LOG2E * V.T).astype(jnp.bfloat16)

    @pl.when(l == 0)
    def _layer1_tile():
        adj_t = adjb_s[pl.ds(r0, _TILE_R), :]
        for hd in range(_NHEADS):
            nu = cu1_s[pl.ds(r0, _TILE_R), hd:hd + 1]
            nv = cv1_s[hd:hd + 1, :]
            s = nu + nv
            arg = jnp.minimum(s, _ALPHA * s)
            e = jnp.exp2(arg) * adj_t
            res = jnp.dot(e, h1_s[hd], preferred_element_type=jnp.float32)
            hp = res[:, :_NHID]
            rowsum = res[:, _NHID:_NHID + 1]
            x1_s[pl.ds(r0, _TILE_R), hd * _NHID:(hd + 1) * _NHID] = (
                _elu(hp / rowsum))

    @pl.when(l == 1)
    def _layer2_tile():
        adj_t = adjb_s[pl.ds(r0, _TILE_R), :]
        nu = cu2_s[pl.ds(r0, _TILE_R), :]
        nv = cv2_s[...]
        s = nu + nv
        arg = jnp.minimum(s, _ALPHA * s)
        e = jnp.exp2(arg) * adj_t
        res = jnp.dot(e, h2_s[...], preferred_element_type=jnp.float32)
        hp = res[:, :_NCLASS]
        rowsum = res[:, _NCLASS:_NCLASS + 1]
        y = _elu(hp / rowsum)
        m = jnp.max(y, axis=1, keepdims=True)
        z = y - m
        lse = jnp.log(jnp.sum(jnp.exp(z), axis=1, keepdims=True))
        out_ref[...] = z - lse


def kernel(T, adj, W_heads, a_heads, W_out, a_out):
    f32 = jnp.float32
    bf16 = jnp.bfloat16

    out = pl.pallas_call(
        _gat_kernel,
        grid=(2, _NTILES),
        in_specs=[
            pl.BlockSpec((_N, _N), lambda l, t: (0, 0)),
            pl.BlockSpec((_N, _NFEAT), lambda l, t: (0, 0)),
            pl.BlockSpec((_NHEADS, _NFEAT, _NHID), lambda l, t: (0, 0, 0)),
            pl.BlockSpec((_NHEADS, 1, 2 * _NHID), lambda l, t: (0, 0, 0)),
            pl.BlockSpec((_NHEADS * _NHID, _NCLASS), lambda l, t: (0, 0)),
            pl.BlockSpec((1, 2 * _NCLASS), lambda l, t: (0, 0)),
        ],
        out_specs=pl.BlockSpec((_TILE_R, _NCLASS), lambda l, t: (t, 0)),
        out_shape=jax.ShapeDtypeStruct((_N, _NCLASS), f32),
        scratch_shapes=[
            pltpu.VMEM((_N, _N), bf16),
            pltpu.VMEM((_NHEADS, _N, _NHID + 1), bf16),
            pltpu.VMEM((_N, _NHEADS), bf16),
            pltpu.VMEM((_NHEADS, _N), bf16),
            pltpu.VMEM((_N, _NHEADS * _NHID), f32),
            pltpu.VMEM((_N, _NCLASS + 1), bf16),
            pltpu.VMEM((_N, 1), bf16),
            pltpu.VMEM((1, _N), bf16),
        ],
        compiler_params=pltpu.CompilerParams(
            dimension_semantics=("arbitrary", "arbitrary")),
    )(adj, T, W_heads, a_heads, W_out, a_out)
    return out


# speedup vs baseline: 1.5675x; 1.1088x over previous
import jax
import jax.numpy as jnp
from jax.experimental import pallas as pl
from jax.experimental.pallas import tpu as pltpu

_N = 2048
_NFEAT = 128
_NHID = 8
_NCLASS = 32
_NHEADS = 8
_ALPHA = 0.2
_LOG2E = 1.4426950408889634
_TILE_R = 512
_NTILES = _N // _TILE_R


def _elu(x):
    return jnp.where(x > 0, x, jnp.exp(x) - 1.0)


def _gat_kernel(adj_ref, T_ref, Wh_ref, ah_ref, Wo_ref, ao_ref, out_ref,
                adjb_s, h1_s, cu1_s, cv1_s, x1_s, h2_s, cu2_s, cv2_s):
    l = pl.program_id(0)
    t = pl.program_id(1)
    r0 = t * _TILE_R

    @pl.when(jnp.logical_and(l == 0, t == 0))
    def _prep1():
        adjb_s[...] = adj_ref[...].astype(jnp.bfloat16)
        Tm = T_ref[...]
        ones = jnp.ones((_N, 1), dtype=jnp.bfloat16)
        for hd in range(_NHEADS):
            h = jnp.dot(Tm, Wh_ref[hd], preferred_element_type=jnp.float32,
                        precision=jax.lax.Precision.HIGHEST)
            h1_s[hd] = jnp.concatenate([h.astype(jnp.bfloat16), ones], axis=1)
        a1 = ah_ref[:, 0, :_NHID]
        a2 = ah_ref[:, 0, _NHID:]
        W1 = jnp.sum(Wh_ref[...] * a1[:, None, :], axis=2).T
        W2 = jnp.sum(Wh_ref[...] * a2[:, None, :], axis=2).T
        U = jnp.dot(Tm, W1, preferred_element_type=jnp.float32,
                    precision=jax.lax.Precision.HIGHEST)
        V = jnp.dot(Tm, W2, preferred_element_type=jnp.float32,
                    precision=jax.lax.Precision.HIGHEST)
        cu1_s[...] = (-_LOG2E * U).astype(jnp.bfloat16)
        cv1_s[...] = (-_LOG2E * V.T).astype(jnp.bfloat16)

    @pl.when(jnp.logical_and(l == 1, t == 0))
    def _prep2():
        xm = x1_s[...]
        h = jnp.dot(xm, Wo_ref[...], preferred_element_type=jnp.float32,
                    precision=jax.lax.Precision.HIGHEST)
        ones = jnp.ones((_N, 1), dtype=jnp.bfloat16)
        h2_s[...] = jnp.concatenate([h.astype(jnp.bfloat16), ones], axis=1)
        a1 = ao_ref[0:1, :_NCLASS]
        a2 = ao_ref[0:1, _NCLASS:]
        W1 = jnp.sum(Wo_ref[...] * a1, axis=1, keepdims=True)
        W2 = jnp.sum(Wo_ref[...] * a2, axis=1, keepdims=True)
        U = jnp.dot(xm, W1, preferred_element_type=jnp.float32,
                    precision=jax.lax.Precision.HIGHEST)
        V = jnp.dot(xm, W2, preferred_element_type=jnp.float32,
                    precision=jax.lax.Precision.HIGHEST)
        cu2_s[...] = (-_LOG2E * U).astype(jnp.bfloat16)
        cv2_s[...] = (-_LOG2E * V.T).astype(jnp.bfloat16)

    @pl.when(l == 0)
    def _layer1_tile():
        adj_t = adjb_s[pl.ds(r0, _TILE_R), :]
        for hd in range(_NHEADS):
            nu = cu1_s[pl.ds(r0, _TILE_R), hd:hd + 1]
            nv = cv1_s[hd:hd + 1, :]
            s = nu + nv
            arg = jnp.minimum(s, _ALPHA * s)
            e = jnp.exp2(arg) * adj_t
            res = jnp.dot(e, h1_s[hd], preferred_element_type=jnp.float32)
            hp = res[:, :_NHID]
            rowsum = res[:, _NHID:_NHID + 1]
            x1_s[pl.ds(r0, _TILE_R), hd * _NHID:(hd + 1) * _NHID] = (
                _elu(hp / rowsum))

    @pl.when(l == 1)
    def _layer2_tile():
        adj_t = adjb_s[pl.ds(r0, _TILE_R), :]
        nu = cu2_s[pl.ds(r0, _TILE_R), :]
        nv = cv2_s[...]
        s = nu + nv
        arg = jnp.minimum(s, _ALPHA * s)
        e = jnp.exp2(arg) * adj_t
        res = jnp.dot(e, h2_s[...], preferred_element_type=jnp.float32)
        hp = res[:, :_NCLASS]
        rowsum = res[:, _NCLASS:_NCLASS + 1]
        y = _elu(hp / rowsum)
        m = jnp.max(y, axis=1, keepdims=True)
        z = y - m
        lse = jnp.log(jnp.sum(jnp.exp(z), axis=1, keepdims=True))
        out_ref[...] = z - lse


def kernel(T, adj, W_heads, a_heads, W_out, a_out):
    f32 = jnp.float32
    bf16 = jnp.bfloat16

    out = pl.pallas_call(
        _gat_kernel,
        grid=(2, _NTILES),
        in_specs=[
            pl.BlockSpec((_N, _N), lambda l, t: (0, 0)),
            pl.BlockSpec((_N, _NFEAT), lambda l, t: (0, 0)),
            pl.BlockSpec((_NHEADS, _NFEAT, _NHID), lambda l, t: (0, 0, 0)),
            pl.BlockSpec((_NHEADS, 1, 2 * _NHID), lambda l, t: (0, 0, 0)),
            pl.BlockSpec((_NHEADS * _NHID, _NCLASS), lambda l, t: (0, 0)),
            pl.BlockSpec((1, 2 * _NCLASS), lambda l, t: (0, 0)),
        ],
        out_specs=pl.BlockSpec((_TILE_R, _NCLASS), lambda l, t: (t, 0)),
        out_shape=jax.ShapeDtypeStruct((_N, _NCLASS), f32),
        scratch_shapes=[
            pltpu.VMEM((_N, _N), bf16),
            pltpu.VMEM((_NHEADS, _N, _NHID + 1), bf16),
            pltpu.VMEM((_N, _NHEADS), bf16),
            pltpu.VMEM((_NHEADS, _N), bf16),
            pltpu.VMEM((_N, _NHEADS * _NHID), f32),
            pltpu.VMEM((_N, _NCLASS + 1), bf16),
            pltpu.VMEM((_N, 1), bf16),
            pltpu.VMEM((1, _N), bf16),
        ],
        compiler_params=pltpu.CompilerParams(
            dimension_semantics=("arbitrary", "arbitrary")),
    )(adj, T, W_heads, a_heads, W_out, a_out)
    return out


# speedup vs baseline: 1.5855x; 1.0115x over previous
import jax
import jax.numpy as jnp
from jax.experimental import pallas as pl
from jax.experimental.pallas import tpu as pltpu

_N = 2048
_NFEAT = 128
_NHID = 8
_NCLASS = 32
_NHEADS = 8
_ALPHA = 0.2
_LOG2E = 1.4426950408889634
_TILE_R = 1024
_NTILES = _N // _TILE_R


def _elu(x):
    return jnp.where(x > 0, x, jnp.exp(x) - 1.0)


def _gat_kernel(adj_ref, T_ref, Wh_ref, ah_ref, Wo_ref, ao_ref, out_ref,
                adjb_s, h1_s, cu1_s, cv1_s, x1_s, h2_s, cu2_s, cv2_s):
    l = pl.program_id(0)
    t = pl.program_id(1)
    r0 = t * _TILE_R

    @pl.when(jnp.logical_and(l == 0, t == 0))
    def _prep1():
        adjb_s[...] = adj_ref[...].astype(jnp.bfloat16)
        Tm = T_ref[...]
        ones = jnp.ones((_N, 1), dtype=jnp.bfloat16)
        for hd in range(_NHEADS):
            h = jnp.dot(Tm, Wh_ref[hd], preferred_element_type=jnp.float32,
                        precision=jax.lax.Precision.HIGHEST)
            h1_s[hd] = jnp.concatenate([h.astype(jnp.bfloat16), ones], axis=1)
        a1 = ah_ref[:, 0, :_NHID]
        a2 = ah_ref[:, 0, _NHID:]
        W1 = jnp.sum(Wh_ref[...] * a1[:, None, :], axis=2).T
        W2 = jnp.sum(Wh_ref[...] * a2[:, None, :], axis=2).T
        U = jnp.dot(Tm, W1, preferred_element_type=jnp.float32,
                    precision=jax.lax.Precision.HIGHEST)
        V = jnp.dot(Tm, W2, preferred_element_type=jnp.float32,
                    precision=jax.lax.Precision.HIGHEST)
        cu1_s[...] = (-_LOG2E * U).astype(jnp.bfloat16)
        cv1_s[...] = (-_LOG2E * V.T).astype(jnp.bfloat16)

    @pl.when(jnp.logical_and(l == 1, t == 0))
    def _prep2():
        xm = x1_s[...]
        h = jnp.dot(xm, Wo_ref[...], preferred_element_type=jnp.float32,
                    precision=jax.lax.Precision.HIGHEST)
        ones = jnp.ones((_N, 1), dtype=jnp.bfloat16)
        h2_s[...] = jnp.concatenate([h.astype(jnp.bfloat16), ones], axis=1)
        a1 = ao_ref[0:1, :_NCLASS]
        a2 = ao_ref[0:1, _NCLASS:]
        W1 = jnp.sum(Wo_ref[...] * a1, axis=1, keepdims=True)
        W2 = jnp.sum(Wo_ref[...] * a2, axis=1, keepdims=True)
        U = jnp.dot(xm, W1, preferred_element_type=jnp.float32,
                    precision=jax.lax.Precision.HIGHEST)
        V = jnp.dot(xm, W2, preferred_element_type=jnp.float32,
                    precision=jax.lax.Precision.HIGHEST)
        cu2_s[...] = (-_LOG2E * U).astype(jnp.bfloat16)
        cv2_s[...] = (-_LOG2E * V.T).astype(jnp.bfloat16)

    @pl.when(l == 0)
    def _layer1_tile():
        adj_t = adjb_s[pl.ds(r0, _TILE_R), :]
        for hd in range(_NHEADS):
            nu = cu1_s[pl.ds(r0, _TILE_R), hd:hd + 1]
            nv = cv1_s[hd:hd + 1, :]
            s = nu + nv
            arg = jnp.minimum(s, _ALPHA * s)
            e = jnp.exp2(arg) * adj_t
            res = jnp.dot(e, h1_s[hd], preferred_element_type=jnp.float32)
            hp = res[:, :_NHID]
            rowsum = res[:, _NHID:_NHID + 1]
            x1_s[pl.ds(r0, _TILE_R), hd * _NHID:(hd + 1) * _NHID] = (
                _elu(hp / rowsum))

    @pl.when(l == 1)
    def _layer2_tile():
        adj_t = adjb_s[pl.ds(r0, _TILE_R), :]
        nu = cu2_s[pl.ds(r0, _TILE_R), :]
        nv = cv2_s[...]
        s = nu + nv
        arg = jnp.minimum(s, _ALPHA * s)
        e = jnp.exp2(arg) * adj_t
        res = jnp.dot(e, h2_s[...], preferred_element_type=jnp.float32)
        hp = res[:, :_NCLASS]
        rowsum = res[:, _NCLASS:_NCLASS + 1]
        y = _elu(hp / rowsum)
        m = jnp.max(y, axis=1, keepdims=True)
        z = y - m
        lse = jnp.log(jnp.sum(jnp.exp(z), axis=1, keepdims=True))
        out_ref[...] = z - lse


def kernel(T, adj, W_heads, a_heads, W_out, a_out):
    f32 = jnp.float32
    bf16 = jnp.bfloat16

    out = pl.pallas_call(
        _gat_kernel,
        grid=(2, _NTILES),
        in_specs=[
            pl.BlockSpec((_N, _N), lambda l, t: (0, 0)),
            pl.BlockSpec((_N, _NFEAT), lambda l, t: (0, 0)),
            pl.BlockSpec((_NHEADS, _NFEAT, _NHID), lambda l, t: (0, 0, 0)),
            pl.BlockSpec((_NHEADS, 1, 2 * _NHID), lambda l, t: (0, 0, 0)),
            pl.BlockSpec((_NHEADS * _NHID, _NCLASS), lambda l, t: (0, 0)),
            pl.BlockSpec((1, 2 * _NCLASS), lambda l, t: (0, 0)),
        ],
        out_specs=pl.BlockSpec((_TILE_R, _NCLASS), lambda l, t: (t, 0)),
        out_shape=jax.ShapeDtypeStruct((_N, _NCLASS), f32),
        scratch_shapes=[
            pltpu.VMEM((_N, _N), bf16),
            pltpu.VMEM((_NHEADS, _N, _NHID + 1), bf16),
            pltpu.VMEM((_N, _NHEADS), bf16),
            pltpu.VMEM((_NHEADS, _N), bf16),
            pltpu.VMEM((_N, _NHEADS * _NHID), f32),
            pltpu.VMEM((_N, _NCLASS + 1), bf16),
            pltpu.VMEM((_N, 1), bf16),
            pltpu.VMEM((1, _N), bf16),
        ],
        compiler_params=pltpu.CompilerParams(
            dimension_semantics=("arbitrary", "arbitrary")),
    )(adj, T, W_heads, a_heads, W_out, a_out)
    return out
